# baseline (speedup 1.0000x reference)
import functools

import jax
import jax.numpy as jnp
from jax import lax
from jax.experimental import pallas as pl
from jax.experimental.pallas import tpu as pltpu
from jax.experimental.pallas import tpu_sc as plsc

N = 10000
E = 320000
D = 128
H = 16
OUT = 128

NC = 2
NS = 16
NW = NC * NS
CHUNK = 128
EROWS = E // CHUNK
RPT_E = EROWS // NW
XTRA = EROWS - RPT_E * NW
NBUF = 13
NOUTER = RPT_E // NBUF
HBUF = 13
HOUTER = RPT_E // HBUF
NPAD = 10240
RPT_N = NPAD // NS
ZROWS = 128

_f32 = jnp.float32


def _sc_mesh():
    return plsc.VectorSubcoreMesh(core_axis_name="c", subcore_axis_name="s")


_SC_PARAMS = pltpu.CompilerParams(use_tc_tiling_on_sc=False)


def _sc_hist(ei):
    @functools.partial(
        pl.kernel,
        out_type=jax.ShapeDtypeStruct((NC, NPAD, H), _f32),
        mesh=_sc_mesh(),
        compiler_params=_SC_PARAMS,
        scratch_types=[
            pltpu.VMEM((RPT_E, CHUNK), jnp.int32),
            pltpu.VMEM((1, CHUNK), jnp.int32),
            pltpu.VMEM((CHUNK, H), _f32),
            pltpu.VMEM((ZROWS, H), _f32),
            pltpu.VMEM_SHARED((NPAD, H), _f32),
            pltpu.SemaphoreType.DMA,
        ],
    )
    def hist_kernel(ei_hbm, out_hbm, dst_v, xdst_v, ones_v, zbuf, acc_sh, hsem):
        core = lax.axis_index("c")
        sub = lax.axis_index("s")
        wid = core * NS + sub

        @pl.loop(0, ZROWS)
        def _(i):
            zbuf[i, :] = jnp.zeros((H,), _f32)

        @pl.loop(0, CHUNK)
        def _(i):
            ones_v[i, :] = jnp.ones((H,), _f32)

        pltpu.sync_copy(ei_hbm.at[1, pl.ds(wid * RPT_E, RPT_E)], dst_v)

        @pl.loop(0, RPT_N // ZROWS)
        def _(j):
            pltpu.sync_copy(zbuf, acc_sh.at[pl.ds(sub * RPT_N + j * ZROWS, ZROWS)])

        plsc.subcore_barrier()

        @pl.loop(0, HOUTER)
        def _(g):
            base = g * HBUF
            for b in range(HBUF):
                pltpu.async_copy(ones_v, acc_sh.at[dst_v.at[base + b]], hsem,
                                 add=True)
            for b in range(HBUF):
                pltpu.make_async_copy(ones_v, acc_sh.at[dst_v.at[base + b]],
                                      hsem).wait()

        @pl.when(wid < XTRA)
        def _():
            pltpu.sync_copy(ei_hbm.at[1, pl.ds(NW * RPT_E + wid, 1)], xdst_v)
            pltpu.sync_copy(ones_v, acc_sh.at[xdst_v.at[0]], add=True)

        plsc.subcore_barrier()
        pltpu.sync_copy(acc_sh.at[pl.ds(sub * RPT_N, RPT_N)],
                        out_hbm.at[core, pl.ds(sub * RPT_N, RPT_N)])

    return hist_kernel(ei)


def _sc_segment_sum(ei, y):
    @functools.partial(
        pl.kernel,
        out_type=jax.ShapeDtypeStruct((NC, NPAD, H), _f32),
        mesh=_sc_mesh(),
        compiler_params=_SC_PARAMS,
        scratch_types=[
            pltpu.VMEM((RPT_E, CHUNK), jnp.int32),
            pltpu.VMEM((RPT_E, CHUNK), jnp.int32),
            pltpu.VMEM((1, CHUNK), jnp.int32),
            pltpu.VMEM((1, CHUNK), jnp.int32),
            pltpu.VMEM((NBUF, CHUNK, H), _f32),
            pltpu.VMEM((ZROWS, H), _f32),
            pltpu.VMEM_SHARED((NPAD, H), _f32),
            pltpu.VMEM_SHARED((NPAD, H), _f32),
            pltpu.SemaphoreType.DMA((NBUF,)),
            pltpu.SemaphoreType.DMA((NBUF,)),
        ],
    )
    def seg_kernel(ei_hbm, y_hbm, out_hbm,
                   src_v, dst_v, xsrc_v, xdst_v, rows_v, zbuf, acc_sh, y_sh,
                   gsem, ssem):
        core = lax.axis_index("c")
        sub = lax.axis_index("s")
        wid = core * NS + sub

        @pl.loop(0, ZROWS)
        def _(i):
            zbuf[i, :] = jnp.zeros((H,), _f32)

        pltpu.sync_copy(ei_hbm.at[0, pl.ds(wid * RPT_E, RPT_E)], src_v)
        pltpu.sync_copy(ei_hbm.at[1, pl.ds(wid * RPT_E, RPT_E)], dst_v)

        pltpu.sync_copy(y_hbm.at[pl.ds(sub * RPT_N, RPT_N)],
                        y_sh.at[pl.ds(sub * RPT_N, RPT_N)])

        @pl.loop(0, RPT_N // ZROWS)
        def _(j):
            pltpu.sync_copy(zbuf, acc_sh.at[pl.ds(sub * RPT_N + j * ZROWS, ZROWS)])

        plsc.subcore_barrier()

        def gstart(ci, b):
            pltpu.async_copy(y_sh.at[src_v.at[ci]], rows_v.at[b], gsem.at[b])

        def gwait(ci, b):
            pltpu.make_async_copy(y_sh.at[src_v.at[ci]], rows_v.at[b],
                                  gsem.at[b]).wait()

        def sstart(ci, b):
            pltpu.async_copy(rows_v.at[b], acc_sh.at[dst_v.at[ci]],
                             ssem.at[b], add=True)

        def swait(ci, b):
            pltpu.make_async_copy(rows_v.at[b], acc_sh.at[dst_v.at[ci]],
                                  ssem.at[b]).wait()

        for b in range(NBUF):
            gstart(b, b)

        @pl.loop(0, NOUTER)
        def _(g):
            base = g * NBUF
            for b in range(NBUF):
                gwait(base + b, b)
                sstart(base + b, b)

            @pl.when(g < NOUTER - 1)
            def _():
                for b in range(NBUF):
                    swait(base + b, b)
                    gstart(base + NBUF + b, b)

        for b in range(NBUF):
            swait(RPT_E - NBUF + b, b)

        @pl.when(wid < XTRA)
        def _():
            pltpu.sync_copy(ei_hbm.at[0, pl.ds(NW * RPT_E + wid, 1)], xsrc_v)
            pltpu.sync_copy(ei_hbm.at[1, pl.ds(NW * RPT_E + wid, 1)], xdst_v)
            pltpu.sync_copy(y_hbm.at[xsrc_v.at[0]], rows_v.at[0])
            pltpu.sync_copy(rows_v.at[0], acc_sh.at[xdst_v.at[0]], add=True)

        plsc.subcore_barrier()
        pltpu.sync_copy(acc_sh.at[pl.ds(sub * RPT_N, RPT_N)],
                        out_hbm.at[core, pl.ds(sub * RPT_N, RPT_N)])

    return seg_kernel(ei, y)


NPV = NPAD * H // 128


def _matmul1_body(x_ref, w_ref, o_ref):
    o_ref[pl.ds(0, N), :] = jnp.dot(x_ref[...], w_ref[...],
                                    preferred_element_type=_f32,
                                    precision=jax.lax.Precision.HIGHEST)
    o_ref[pl.ds(N, NPAD - N), :] = jnp.zeros((NPAD - N, H), _f32)


def _scale_body(p_ref, xw_ref, y_ref, dinv_ref):
    deg = p_ref[0] + p_ref[1] + 1.0
    dinv = lax.rsqrt(deg)
    dinv_ref[...] = dinv
    y_ref[...] = xw_ref[...] * dinv


def _post1_body(u_ref, dinv_ref, xw_ref, b1_ref, h_ref, y2_ref):
    dinv = dinv_ref[...]
    agg = dinv * (u_ref[0] + u_ref[1]) + dinv * dinv * xw_ref[...]
    h = jnp.maximum(agg + b1_ref[...], 0.0)
    h_ref[...] = h
    y2_ref[...] = h * dinv


def _final_body(u_ref, dinv_ref, h_ref, wb2_ref, b2_ref, o_ref):
    dinv = dinv_ref[...]
    agg = dinv * (u_ref[0] + u_ref[1]) + dinv * dinv * h_ref[...]
    o = jnp.dot(agg, wb2_ref[...],
                preferred_element_type=_f32,
                precision=jax.lax.Precision.HIGHEST)
    for j in range(8):
        seg = o[:, 128 * j:128 * (j + 1)] + b2_ref[...]
        m = jnp.max(seg, axis=-1, keepdims=True)
        s = seg - m
        lse = jnp.log(jnp.sum(jnp.exp(s), axis=-1, keepdims=True))
        o_ref[:, 128 * j:128 * (j + 1)] = s - lse


def kernel(x, edge_index, W1, b1, W2, b2):
    ei = edge_index.reshape(2, EROWS, CHUNK)
    b1t = jnp.tile(b1.reshape(1, H), (1, 8))
    wb2 = jnp.kron(jnp.eye(8, dtype=_f32), W2)

    hist = _sc_hist(ei)
    xw1 = pl.pallas_call(
        _matmul1_body,
        out_shape=jax.ShapeDtypeStruct((NPAD, H), _f32),
    )(x, W1)

    hist_v = hist.reshape(NC, NPV, 128)
    xw_v = xw1.reshape(NPV, 128)

    y1_v, dinv_v = pl.pallas_call(
        _scale_body,
        out_shape=(jax.ShapeDtypeStruct((NPV, 128), _f32),
                   jax.ShapeDtypeStruct((NPV, 128), _f32)),
    )(hist_v, xw_v)

    u1 = _sc_segment_sum(ei, y1_v.reshape(NPAD, H))

    h_v, y2_v = pl.pallas_call(
        _post1_body,
        out_shape=(jax.ShapeDtypeStruct((NPV, 128), _f32),
                   jax.ShapeDtypeStruct((NPV, 128), _f32)),
    )(u1.reshape(NC, NPV, 128), dinv_v, xw_v, b1t)

    u2 = _sc_segment_sum(ei, y2_v.reshape(NPAD, H))

    out_big = pl.pallas_call(
        _final_body,
        out_shape=jax.ShapeDtypeStruct((NPV, 1024), _f32),
    )(u2.reshape(NC, NPV, 128), dinv_v, h_v, wb2, b2.reshape(1, OUT))
    return out_big.reshape(NPAD, OUT)[:N]


# speedup vs baseline: 1.0292x; 1.0292x over previous
import functools

import jax
import jax.numpy as jnp
from jax import lax
from jax.experimental import pallas as pl
from jax.experimental.pallas import tpu as pltpu
from jax.experimental.pallas import tpu_sc as plsc

N = 10000
E = 320000
D = 128
H = 16
OUT = 128

NC = 2
NS = 16
NW = NC * NS
CHUNK = 128
EROWS = E // CHUNK
RPT_E = EROWS // NW
XTRA = EROWS - RPT_E * NW
NBUF = 13
NOUTER = RPT_E // NBUF
HBUF = 13
HOUTER = RPT_E // HBUF
NPAD = 10240
RPT_N = NPAD // NS
ZROWS = 128

_f32 = jnp.float32


def _sc_mesh():
    return plsc.VectorSubcoreMesh(core_axis_name="c", subcore_axis_name="s")


_SC_PARAMS = pltpu.CompilerParams(use_tc_tiling_on_sc=False)


def _sc_hist(ei):
    @functools.partial(
        pl.kernel,
        out_type=jax.ShapeDtypeStruct((NC, NPAD, H), _f32),
        mesh=_sc_mesh(),
        compiler_params=_SC_PARAMS,
        scratch_types=[
            pltpu.VMEM((RPT_E, CHUNK), jnp.int32),
            pltpu.VMEM((1, CHUNK), jnp.int32),
            pltpu.VMEM((CHUNK, H), _f32),
            pltpu.VMEM((ZROWS, H), _f32),
            pltpu.VMEM_SHARED((NPAD, H), _f32),
            pltpu.SemaphoreType.DMA,
        ],
    )
    def hist_kernel(ei_hbm, out_hbm, dst_v, xdst_v, ones_v, zbuf, acc_sh, hsem):
        core = lax.axis_index("c")
        sub = lax.axis_index("s")
        wid = core * NS + sub

        @pl.loop(0, ZROWS)
        def _(i):
            zbuf[i, :] = jnp.zeros((H,), _f32)

        @pl.loop(0, CHUNK)
        def _(i):
            ones_v[i, :] = jnp.ones((H,), _f32)

        pltpu.sync_copy(ei_hbm.at[1, pl.ds(wid * RPT_E, RPT_E)], dst_v)

        @pl.loop(0, RPT_N // ZROWS)
        def _(j):
            pltpu.sync_copy(zbuf, acc_sh.at[pl.ds(sub * RPT_N + j * ZROWS, ZROWS)])

        plsc.subcore_barrier()

        @pl.loop(0, HOUTER)
        def _(g):
            base = g * HBUF
            for b in range(HBUF):
                pltpu.async_copy(ones_v, acc_sh.at[dst_v.at[base + b]], hsem,
                                 add=True)
            for b in range(HBUF):
                pltpu.make_async_copy(ones_v, acc_sh.at[dst_v.at[base + b]],
                                      hsem).wait()

        @pl.when(wid < XTRA)
        def _():
            pltpu.sync_copy(ei_hbm.at[1, pl.ds(NW * RPT_E + wid, 1)], xdst_v)
            pltpu.sync_copy(ones_v, acc_sh.at[xdst_v.at[0]], add=True)

        plsc.subcore_barrier()
        pltpu.sync_copy(acc_sh.at[pl.ds(sub * RPT_N, RPT_N)],
                        out_hbm.at[core, pl.ds(sub * RPT_N, RPT_N)])

    return hist_kernel(ei)


def _sc_segment_sum(ei, y):
    @functools.partial(
        pl.kernel,
        out_type=jax.ShapeDtypeStruct((NC, NPAD, H), _f32),
        mesh=_sc_mesh(),
        compiler_params=_SC_PARAMS,
        scratch_types=[
            pltpu.VMEM((RPT_E, CHUNK), jnp.int32),
            pltpu.VMEM((RPT_E, CHUNK), jnp.int32),
            pltpu.VMEM((1, CHUNK), jnp.int32),
            pltpu.VMEM((1, CHUNK), jnp.int32),
            pltpu.VMEM((NBUF, CHUNK, H), _f32),
            pltpu.VMEM((ZROWS, H), _f32),
            pltpu.VMEM_SHARED((NPAD, H), _f32),
            pltpu.SemaphoreType.DMA((NBUF,)),
            pltpu.SemaphoreType.DMA((NBUF,)),
        ],
    )
    def seg_kernel(ei_hbm, y_hbm, out_hbm,
                   src_v, dst_v, xsrc_v, xdst_v, rows_v, zbuf, acc_sh,
                   gsem, ssem):
        core = lax.axis_index("c")
        sub = lax.axis_index("s")
        wid = core * NS + sub

        @pl.loop(0, ZROWS)
        def _(i):
            zbuf[i, :] = jnp.zeros((H,), _f32)

        pltpu.sync_copy(ei_hbm.at[0, pl.ds(wid * RPT_E, RPT_E)], src_v)
        pltpu.sync_copy(ei_hbm.at[1, pl.ds(wid * RPT_E, RPT_E)], dst_v)

        @pl.loop(0, RPT_N // ZROWS)
        def _(j):
            pltpu.sync_copy(zbuf, acc_sh.at[pl.ds(sub * RPT_N + j * ZROWS, ZROWS)])

        plsc.subcore_barrier()

        def gstart(ci, b):
            pltpu.async_copy(y_hbm.at[src_v.at[ci]], rows_v.at[b], gsem.at[b])

        def gwait(ci, b):
            pltpu.make_async_copy(y_hbm.at[src_v.at[ci]], rows_v.at[b],
                                  gsem.at[b]).wait()

        def sstart(ci, b):
            pltpu.async_copy(rows_v.at[b], acc_sh.at[dst_v.at[ci]],
                             ssem.at[b], add=True)

        def swait(ci, b):
            pltpu.make_async_copy(rows_v.at[b], acc_sh.at[dst_v.at[ci]],
                                  ssem.at[b]).wait()

        for b in range(NBUF):
            gstart(b, b)

        @pl.loop(0, NOUTER)
        def _(g):
            base = g * NBUF
            for b in range(NBUF):
                gwait(base + b, b)
                sstart(base + b, b)

            @pl.when(g < NOUTER - 1)
            def _():
                for b in range(NBUF):
                    swait(base + b, b)
                    gstart(base + NBUF + b, b)

        for b in range(NBUF):
            swait(RPT_E - NBUF + b, b)

        @pl.when(wid < XTRA)
        def _():
            pltpu.sync_copy(ei_hbm.at[0, pl.ds(NW * RPT_E + wid, 1)], xsrc_v)
            pltpu.sync_copy(ei_hbm.at[1, pl.ds(NW * RPT_E + wid, 1)], xdst_v)
            pltpu.sync_copy(y_hbm.at[xsrc_v.at[0]], rows_v.at[0])
            pltpu.sync_copy(rows_v.at[0], acc_sh.at[xdst_v.at[0]], add=True)

        plsc.subcore_barrier()
        pltpu.sync_copy(acc_sh.at[pl.ds(sub * RPT_N, RPT_N)],
                        out_hbm.at[core, pl.ds(sub * RPT_N, RPT_N)])

    return seg_kernel(ei, y)


NPV = NPAD * H // 128


def _matmul1_body(x_ref, w_ref, o_ref):
    o_ref[pl.ds(0, N), :] = jnp.dot(x_ref[...], w_ref[...],
                                    preferred_element_type=_f32,
                                    precision=jax.lax.Precision.HIGHEST)
    o_ref[pl.ds(N, NPAD - N), :] = jnp.zeros((NPAD - N, H), _f32)


def _scale_body(p_ref, xw_ref, y_ref, dinv_ref):
    deg = p_ref[0] + p_ref[1] + 1.0
    dinv = lax.rsqrt(deg)
    dinv_ref[...] = dinv
    y_ref[...] = xw_ref[...] * dinv


def _post1_body(u_ref, dinv_ref, xw_ref, b1_ref, h_ref, y2_ref):
    dinv = dinv_ref[...]
    agg = dinv * (u_ref[0] + u_ref[1]) + dinv * dinv * xw_ref[...]
    h = jnp.maximum(agg + b1_ref[...], 0.0)
    h_ref[...] = h
    y2_ref[...] = h * dinv


def _final_body(u_ref, dinv_ref, h_ref, wb2_ref, b2_ref, o_ref):
    dinv = dinv_ref[...]
    agg = dinv * (u_ref[0] + u_ref[1]) + dinv * dinv * h_ref[...]
    o = jnp.dot(agg, wb2_ref[...],
                preferred_element_type=_f32,
                precision=jax.lax.Precision.HIGHEST)
    for j in range(8):
        seg = o[:, 128 * j:128 * (j + 1)] + b2_ref[...]
        m = jnp.max(seg, axis=-1, keepdims=True)
        s = seg - m
        lse = jnp.log(jnp.sum(jnp.exp(s), axis=-1, keepdims=True))
        o_ref[:, 128 * j:128 * (j + 1)] = s - lse


def kernel(x, edge_index, W1, b1, W2, b2):
    ei = edge_index.reshape(2, EROWS, CHUNK)
    b1t = jnp.tile(b1.reshape(1, H), (1, 8))
    wb2 = jnp.kron(jnp.eye(8, dtype=_f32), W2)

    hist = _sc_hist(ei)
    xw1 = pl.pallas_call(
        _matmul1_body,
        out_shape=jax.ShapeDtypeStruct((NPAD, H), _f32),
    )(x, W1)

    hist_v = hist.reshape(NC, NPV, 128)
    xw_v = xw1.reshape(NPV, 128)

    y1_v, dinv_v = pl.pallas_call(
        _scale_body,
        out_shape=(jax.ShapeDtypeStruct((NPV, 128), _f32),
                   jax.ShapeDtypeStruct((NPV, 128), _f32)),
    )(hist_v, xw_v)

    u1 = _sc_segment_sum(ei, y1_v.reshape(NPAD, H))

    h_v, y2_v = pl.pallas_call(
        _post1_body,
        out_shape=(jax.ShapeDtypeStruct((NPV, 128), _f32),
                   jax.ShapeDtypeStruct((NPV, 128), _f32)),
    )(u1.reshape(NC, NPV, 128), dinv_v, xw_v, b1t)

    u2 = _sc_segment_sum(ei, y2_v.reshape(NPAD, H))

    out_big = pl.pallas_call(
        _final_body,
        out_shape=jax.ShapeDtypeStruct((NPV, 1024), _f32),
    )(u2.reshape(NC, NPV, 128), dinv_v, h_v, wb2, b2.reshape(1, OUT))
    return out_big.reshape(NPAD, OUT)[:N]
